# baseline (speedup 1.0000x reference)
import functools

import jax
import jax.numpy as jnp
from jax import lax
from jax.experimental import pallas as pl
from jax.experimental.pallas import tpu as pltpu
from jax.experimental.pallas import tpu_sc as plsc

B = 4096
L = 200
D = 64
NC = 2
NS = 16
NW = NC * NS
BPW = B // NW
CPB = 2
CL = L // CPB

_mesh = plsc.VectorSubcoreMesh(core_axis_name="c", subcore_axis_name="s")


@functools.partial(
    pl.kernel,
    mesh=_mesh,
    compiler_params=pltpu.CompilerParams(use_tc_tiling_on_sc=False),
    out_type=jax.ShapeDtypeStruct((B, D), jnp.float32),
    scratch_types=[
        pltpu.VMEM((BPW * CPB, CL), jnp.int32),
        pltpu.VMEM((CL, D), jnp.float32),
        pltpu.VMEM((CL, D), jnp.float32),
        pltpu.VMEM((BPW, D), jnp.float32),
        pltpu.SemaphoreType.DMA,
        pltpu.SemaphoreType.DMA,
    ],
)
def _emb_pool(x_hbm, table_hbm, out_hbm, idx_v, rows0, rows1, out_v, sem0, sem1):
    wid = lax.axis_index("s") * NC + lax.axis_index("c")
    pltpu.sync_copy(x_hbm.at[pl.ds(wid * BPW * CPB, BPW * CPB)], idx_v)

    def body(b, carry):
        cp0 = pltpu.async_copy(table_hbm.at[idx_v.at[2 * b]], rows0, sem0)
        cp1 = pltpu.async_copy(table_hbm.at[idx_v.at[2 * b + 1]], rows1, sem1)
        cp0.wait()
        cp1.wait()

        def acc_body(r, accs):
            return tuple(
                accs[c] + rows0[r, pl.ds(c * 16, 16)] + rows1[r, pl.ds(c * 16, 16)]
                for c in range(D // 16)
            )

        accs = lax.fori_loop(
            0, CL, acc_body,
            tuple(jnp.zeros((16,), jnp.float32) for _ in range(D // 16)),
        )
        for c in range(D // 16):
            out_v[b, pl.ds(c * 16, 16)] = accs[c] * (1.0 / L)
        return carry

    lax.fori_loop(0, BPW, body, 0)
    pltpu.sync_copy(out_v, out_hbm.at[pl.ds(wid * BPW, BPW)])


def kernel(x, table):
    x2 = x.reshape(B * CPB, CL).astype(jnp.int32)
    return _emb_pool(x2, table)


# speedup vs baseline: 1.1206x; 1.1206x over previous
import functools

import jax
import jax.numpy as jnp
from jax import lax
from jax.experimental import pallas as pl
from jax.experimental.pallas import tpu as pltpu
from jax.experimental.pallas import tpu_sc as plsc

B = 4096
L = 200
D = 64
NC = 2
NS = 16
NW = NC * NS
BPW = B // NW
CPB = 2
CL = L // CPB

_mesh = plsc.VectorSubcoreMesh(core_axis_name="c", subcore_axis_name="s")


NCH = D // 16
UN = 4


@functools.partial(
    pl.kernel,
    mesh=_mesh,
    compiler_params=pltpu.CompilerParams(use_tc_tiling_on_sc=False),
    out_type=jax.ShapeDtypeStruct((B, D), jnp.float32),
    scratch_types=[
        pltpu.VMEM((BPW * CPB, CL), jnp.int32),
        pltpu.VMEM((CL, D), jnp.float32),
        pltpu.VMEM((CL, D), jnp.float32),
        pltpu.VMEM((CL, D), jnp.float32),
        pltpu.VMEM((CL, D), jnp.float32),
        pltpu.VMEM((BPW, D), jnp.float32),
        pltpu.SemaphoreType.DMA,
        pltpu.SemaphoreType.DMA,
        pltpu.SemaphoreType.DMA,
        pltpu.SemaphoreType.DMA,
    ],
)
def _emb_pool(x_hbm, table_hbm, out_hbm, idx_v, ra0, ra1, rb0, rb1, out_v,
              sa0, sa1, sb0, sb1):
    wid = lax.axis_index("s") * NC + lax.axis_index("c")
    pltpu.sync_copy(x_hbm.at[pl.ds(wid * BPW * CPB, BPW * CPB)], idx_v)

    pair_a = ((ra0, sa0), (ra1, sa1))
    pair_b = ((rb0, sb0), (rb1, sb1))

    def descs(elt, pair):
        return [
            pltpu.make_async_copy(table_hbm.at[idx_v.at[2 * elt + k]], buf, sem)
            for k, (buf, sem) in enumerate(pair)
        ]

    def start(elt, pair):
        for d in descs(elt, pair):
            d.start()

    def wait(elt, pair):
        for d in descs(elt, pair):
            d.wait()

    def accumulate(elt, pair):
        bufs = (pair[0][0], pair[1][0])

        def acc_body(j, accs):
            r = j * UN
            new = list(accs)
            for k in range(UN):
                for c in range(NCH):
                    new[c] = new[c] + bufs[0][r + k, pl.ds(c * 16, 16)]
                    new[NCH + c] = new[NCH + c] + bufs[1][r + k, pl.ds(c * 16, 16)]
            return tuple(new)

        accs = lax.fori_loop(
            0, CL // UN, acc_body,
            tuple(jnp.zeros((16,), jnp.float32) for _ in range(2 * NCH)),
        )
        for c in range(NCH):
            out_v[elt, pl.ds(c * 16, 16)] = (accs[c] + accs[NCH + c]) * (1.0 / L)

    start(0, pair_a)

    def outer(i, carry):
        b0 = 2 * i
        start(b0 + 1, pair_b)
        wait(b0, pair_a)
        accumulate(b0, pair_a)
        start(jnp.minimum(b0 + 2, BPW - 1), pair_a)
        wait(b0 + 1, pair_b)
        accumulate(b0 + 1, pair_b)
        return carry

    lax.fori_loop(0, BPW // 2, outer, 0)
    wait(BPW - 1, pair_a)
    pltpu.sync_copy(out_v, out_hbm.at[pl.ds(wid * BPW, BPW)])


def kernel(x, table):
    x2 = x.reshape(B * CPB, CL).astype(jnp.int32)
    return _emb_pool(x2, table)
